# Initial kernel scaffold; baseline (speedup 1.0000x reference)
import functools
import jax
import jax.numpy as jnp
from jax import lax
from jax.experimental import pallas as pl
from jax.experimental.pallas import tpu as pltpu, tpu_sc as plsc

N = 50000
E = 1600000
EPR = 12544
N_PAD = 50048
ZR = N_PAD // 16
BLK = 1000
F32 = jnp.float32
I32 = jnp.int32


def _lrelu(a):
    return jnp.where(a >= 0, a, 0.2 * a)


def _stage_a_body(x_ref, w1t, b1, w2t, b2, afse, afde, afsb, afdb,
                  tab_ref, dst_ref):
    xb = x_ref[...]
    h1 = jnp.maximum(jnp.dot(xb, w1t[...], preferred_element_type=F32)
                     + b1[...], 0.0)
    ctx = jnp.dot(h1, w2t[...], preferred_element_type=F32) + b2[...]
    xenv = jnp.concatenate([xb[:, 0:1], xb[:, 1:2], xb[:, 5:6], ctx], axis=1)
    xbeh = jnp.concatenate([xb[:, 0:1], xb[:, 1:2], xb[:, 2:3], ctx], axis=1)
    ase = jnp.dot(xenv, afse[...], preferred_element_type=F32)
    ade = jnp.dot(xenv, afde[...], preferred_element_type=F32)
    asb = jnp.dot(xbeh, afsb[...], preferred_element_type=F32)
    adb = jnp.dot(xbeh, afdb[...], preferred_element_type=F32)
    me = _lrelu(ase + ade)
    mb = _lrelu(asb + adb)
    z3 = jnp.zeros((xb.shape[0], 3), F32)
    tab_ref[0] = jnp.concatenate([xenv, ase, z3], axis=1)
    tab_ref[1] = jnp.concatenate([xbeh, asb, z3], axis=1)
    z8 = jnp.zeros((xb.shape[0], 8), F32)
    dst_ref[...] = jnp.concatenate([ade, me, adb, mb, z8], axis=1)


def _stage_a(x, w1t, b1, w2t, b2, afse, afde, afsb, afdb):
    grid = (N // BLK,)
    full = lambda a: pl.BlockSpec(a.shape, lambda i: (0,) * a.ndim)
    return pl.pallas_call(
        _stage_a_body,
        grid=grid,
        in_specs=[pl.BlockSpec((BLK, 6), lambda i: (i, 0)),
                  full(w1t), full(b1), full(w2t), full(b2),
                  full(afse), full(afde), full(afsb), full(afdb)],
        out_specs=[pl.BlockSpec((2, BLK, 16), lambda i: (0, i, 0)),
                   pl.BlockSpec((BLK, 16), lambda i: (i, 0))],
        out_shape=[jax.ShapeDtypeStruct((2, N, 16), F32),
                   jax.ShapeDtypeStruct((N, 16), F32)],
    )(x, w1t, b1, w2t, b2, afse, afde, afsb, afdb)


def _pass1_body(sidx2, didx, stab, dtab, zro, out,
                s_idx, d_idx, s_rows, d_rows, payl, accum, sem):
    cid = lax.axis_index("c")
    sid = lax.axis_index("s")
    pltpu.sync_copy(zro.at[pl.ds(sid * ZR, ZR)], accum.at[pl.ds(sid * ZR, ZR)])
    lane = lax.iota(I32, 16)
    ccol = [jnp.full((16,), c, I32) for c in range(24)]
    co = cid * 4
    dcol = [ccol[c] + co for c in range(4)]

    def chunk(i, carry):
        row0 = sid * 784 + i * 8
        pltpu.sync_copy(sidx2.at[cid, pl.ds(row0, 8)], s_idx)
        pltpu.sync_copy(didx.at[pl.ds(row0, 8)], d_idx)
        hs = []
        for j in range(8):
            hs.append(pltpu.async_copy(
                stab.at[s_idx.at[j]], s_rows.at[pl.ds(j * 128, 128)], sem))
            hs.append(pltpu.async_copy(
                dtab.at[d_idx.at[j]], d_rows.at[pl.ds(j * 128, 128)], sem))
        for h in hs:
            h.wait()

        def group(g, c2):
            r = g * 16 + lane
            feat = [plsc.load_gather(s_rows, [r, ccol[c]]) for c in range(11)]
            as0 = plsc.load_gather(s_rows, [r, ccol[11]])
            as1 = plsc.load_gather(s_rows, [r, ccol[12]])
            ad0 = plsc.load_gather(d_rows, [r, dcol[0]])
            ad1 = plsc.load_gather(d_rows, [r, dcol[1]])
            m0 = plsc.load_gather(d_rows, [r, dcol[2]])
            m1 = plsc.load_gather(d_rows, [r, dcol[3]])
            t0 = jnp.exp(_lrelu(as0 + ad0) - m0)
            t1 = jnp.exp(_lrelu(as1 + ad1) - m1)
            plsc.store_scatter(payl, [r, ccol[0]], t0)
            plsc.store_scatter(payl, [r, ccol[1]], t1)
            for c in range(11):
                plsc.store_scatter(payl, [r, ccol[2 + c]], t0 * feat[c])
                plsc.store_scatter(payl, [r, ccol[13 + c]], t1 * feat[c])
            return c2
        lax.fori_loop(0, 64, group, 0)
        for j in range(8):
            pltpu.sync_copy(payl.at[pl.ds(j * 128, 128)],
                            accum.at[d_idx.at[j]], add=True)
        return carry
    plsc.subcore_barrier()
    lax.fori_loop(0, 98, chunk, 0)
    plsc.subcore_barrier()
    pltpu.sync_copy(accum.at[pl.ds(sid * ZR, ZR)],
                    out.at[cid, pl.ds(sid * ZR, ZR)])


def _pass1(sidx2, didx, stab, dtab, zro):
    mesh = plsc.VectorSubcoreMesh(core_axis_name="c", subcore_axis_name="s")
    kern = functools.partial(
        pl.kernel,
        mesh=mesh,
        out_type=jax.ShapeDtypeStruct((2, N_PAD, 24), F32),
        scratch_types=[
            pltpu.VMEM((8, 128), I32),
            pltpu.VMEM((8, 128), I32),
            pltpu.VMEM((1024, 16), F32),
            pltpu.VMEM((1024, 16), F32),
            pltpu.VMEM((1024, 24), F32),
            pltpu.VMEM_SHARED((N_PAD, 24), F32),
            pltpu.SemaphoreType.DMA,
        ])(_pass1_body)
    return kern(sidx2, didx, stab, dtab, zro)


def _stage_b_body(accA, accB, tab, x_ref, Menv, benv, Mbeh, bbeh,
                  e2w, eas, ead, b2w, bas, bad, gw1t, gb1, gw2t, gb2,
                  src2_ref, dst2_ref, aux_ref):
    B = x_ref.shape[0]

    def layer1(acc, xf, M, b):
        den = acc[:, 0:2] + 1.0
        U = acc[:, 2:24] + jnp.concatenate([xf, xf], axis=1)
        d11 = jnp.concatenate(
            [jnp.broadcast_to(den[:, 0:1], (B, 11)),
             jnp.broadcast_to(den[:, 1:2], (B, 11))], axis=1) + 1e-16
        h = jnp.dot(U / d11, M[...], preferred_element_type=F32) + b[...]
        return jnp.where(h > 0, h, jnp.exp(jnp.minimum(h, 0.0)) - 1.0)

    xenv = tab[0][:, 0:11]
    xbeh = tab[1][:, 0:11]
    env_h = layer1(accA[...], xenv, Menv, benv)
    beh_h = layer1(accB[...], xbeh, Mbeh, bbeh)
    xl2e = jnp.dot(env_h, e2w[...], preferred_element_type=F32)
    xl2b = jnp.dot(beh_h, b2w[...], preferred_element_type=F32)
    as2e = xl2e * eas[0, 0]
    ad2e = xl2e * ead[0, 0]
    m2e = _lrelu(as2e + ad2e)
    as2b = xl2b * bas[0, 0]
    ad2b = xl2b * bad[0, 0]
    m2b = _lrelu(as2b + ad2b)
    ctx = tab[0][:, 3:11]
    g1 = jnp.maximum(jnp.dot(ctx, gw1t[...], preferred_element_type=F32)
                     + gb1[...], 0.0)
    g2 = jnp.dot(g1, gw2t[...], preferred_element_type=F32) + gb2[...]
    g2 = g2 - jnp.max(g2, axis=1, keepdims=True)
    eg = jnp.exp(g2)
    gate = eg / jnp.sum(eg, axis=1, keepdims=True)
    xb = x_ref[...]
    z2 = jnp.zeros((B, 2), F32)
    src2_ref[...] = jnp.concatenate(
        [xl2e, as2e, xl2b, as2b, xb[:, 3:4], xb[:, 4:5], ctx, z2], axis=1)
    dst2_ref[...] = jnp.concatenate(
        [ad2e, m2e, ad2b, m2b, jnp.zeros((B, 12), F32)], axis=1)
    aux_ref[...] = jnp.concatenate(
        [gate, xl2e, xl2b, jnp.zeros((B, 3), F32)], axis=1)


def _stage_b(accA, accB, tab2, x, Menv, benv, Mbeh, bbeh,
             e2w, eas, ead, b2w, bas, bad, gw1t, gb1, gw2t, gb2):
    grid = (N // BLK,)
    full = lambda a: pl.BlockSpec(a.shape, lambda i: (0,) * a.ndim)
    args = (accA, accB, tab2, x, Menv, benv, Mbeh, bbeh,
            e2w, eas, ead, b2w, bas, bad, gw1t, gb1, gw2t, gb2)
    return pl.pallas_call(
        _stage_b_body,
        grid=grid,
        in_specs=[pl.BlockSpec((BLK, 24), lambda i: (i, 0)),
                  pl.BlockSpec((BLK, 24), lambda i: (i, 0)),
                  pl.BlockSpec((2, BLK, 16), lambda i: (0, i, 0)),
                  pl.BlockSpec((BLK, 6), lambda i: (i, 0))] +
                 [full(a) for a in args[4:]],
        out_specs=[pl.BlockSpec((BLK, 16), lambda i: (i, 0)),
                   pl.BlockSpec((BLK, 16), lambda i: (i, 0)),
                   pl.BlockSpec((BLK, 8), lambda i: (i, 0))],
        out_shape=[jax.ShapeDtypeStruct((N, 16), F32),
                   jax.ShapeDtypeStruct((N, 16), F32),
                   jax.ShapeDtypeStruct((N, 8), F32)],
    )(*args)


def _pass2_body(sidx, didx, stab, dtab, zro, out,
                s_idx, d_idx, s_rows, d_rows, payl, accum, sem):
    cid = lax.axis_index("c")
    sid = lax.axis_index("s")
    pltpu.sync_copy(zro.at[pl.ds(sid * ZR, ZR)], accum.at[pl.ds(sid * ZR, ZR)])
    lane = lax.iota(I32, 16)
    ccol = [jnp.full((16,), c, I32) for c in range(16)]
    wid = sid * 2 + cid
    zv = jnp.zeros((16,), F32)

    def chunk(i, carry):
        row0 = wid * 392 + i * 8
        pltpu.sync_copy(sidx.at[pl.ds(row0, 8)], s_idx)
        pltpu.sync_copy(didx.at[pl.ds(row0, 8)], d_idx)
        hs = []
        for j in range(8):
            hs.append(pltpu.async_copy(
                stab.at[s_idx.at[j]], s_rows.at[pl.ds(j * 128, 128)], sem))
            hs.append(pltpu.async_copy(
                dtab.at[d_idx.at[j]], d_rows.at[pl.ds(j * 128, 128)], sem))
        for h in hs:
            h.wait()

        def group(g, c2):
            r = g * 16 + lane
            xl2e = plsc.load_gather(s_rows, [r, ccol[0]])
            as2e = plsc.load_gather(s_rows, [r, ccol[1]])
            xl2b = plsc.load_gather(s_rows, [r, ccol[2]])
            as2b = plsc.load_gather(s_rows, [r, ccol[3]])
            ad2e = plsc.load_gather(d_rows, [r, ccol[0]])
            m2e = plsc.load_gather(d_rows, [r, ccol[1]])
            ad2b = plsc.load_gather(d_rows, [r, ccol[2]])
            m2b = plsc.load_gather(d_rows, [r, ccol[3]])
            te = jnp.exp(_lrelu(as2e + ad2e) - m2e)
            tb = jnp.exp(_lrelu(as2b + ad2b) - m2b)
            plsc.store_scatter(payl, [r, ccol[0]], te)
            plsc.store_scatter(payl, [r, ccol[1]], te * xl2e)
            plsc.store_scatter(payl, [r, ccol[2]], tb)
            plsc.store_scatter(payl, [r, ccol[3]], tb * xl2b)
            for c in range(4, 14):
                v = plsc.load_gather(s_rows, [r, ccol[c]])
                plsc.store_scatter(payl, [r, ccol[c]], v)
            plsc.store_scatter(payl, [r, ccol[14]], zv)
            plsc.store_scatter(payl, [r, ccol[15]], zv)
            return c2
        lax.fori_loop(0, 64, group, 0)
        for j in range(8):
            pltpu.sync_copy(payl.at[pl.ds(j * 128, 128)],
                            accum.at[d_idx.at[j]], add=True)
        return carry
    plsc.subcore_barrier()
    lax.fori_loop(0, 49, chunk, 0)
    plsc.subcore_barrier()
    pltpu.sync_copy(accum.at[pl.ds(sid * ZR, ZR)],
                    out.at[cid, pl.ds(sid * ZR, ZR)])


def _pass2(sidx, didx, stab, dtab, zro):
    mesh = plsc.VectorSubcoreMesh(core_axis_name="c", subcore_axis_name="s")
    kern = functools.partial(
        pl.kernel,
        mesh=mesh,
        out_type=jax.ShapeDtypeStruct((2, N_PAD, 16), F32),
        scratch_types=[
            pltpu.VMEM((8, 128), I32),
            pltpu.VMEM((8, 128), I32),
            pltpu.VMEM((1024, 16), F32),
            pltpu.VMEM((1024, 16), F32),
            pltpu.VMEM((1024, 16), F32),
            pltpu.VMEM_SHARED((N_PAD, 16), F32),
            pltpu.SemaphoreType.DMA,
        ])(_pass2_body)
    return kern(sidx, didx, stab, dtab, zro)


def _evolve_body(wg, wiht, bsum, out_ref):
    out_ref[...] = jnp.tanh(
        jnp.dot(wg[...], wiht[...], preferred_element_type=F32) + bsum[...])


def _evolve(wg, wiht, bsum):
    return pl.pallas_call(
        _evolve_body,
        out_shape=jax.ShapeDtypeStruct((1, 100), F32),
    )(wg, wiht, bsum)


def _stage_c_body(a2a, a2b, aux, wnewt, pjw, pjb, eb2, bb2, out_ref):
    a2 = a2a[...] + a2b[...]
    xl2e = aux[:, 3:4]
    xl2b = aux[:, 4:5]
    env_out = (a2[:, 1:2] + xl2e) / (a2[:, 0:1] + 1.0 + 1e-16) + eb2[0, 0]
    beh_out = (a2[:, 3:4] + xl2b) / (a2[:, 2:3] + 1.0 + 1e-16) + bb2[0, 0]
    aggf = a2[:, 4:14]
    ph = jnp.maximum(jnp.dot(aggf, wnewt[...], preferred_element_type=F32),
                     0.0)
    phys_out = jnp.dot(ph, pjw[...], preferred_element_type=F32) + pjb[0, 0]
    out_ref[...] = (aux[:, 0:1] * env_out + aux[:, 1:2] * phys_out
                    + aux[:, 2:3] * beh_out)


def _stage_c(a2a, a2b, aux, wnewt, pjw, pjb, eb2, bb2):
    grid = (N // BLK,)
    full = lambda a: pl.BlockSpec(a.shape, lambda i: (0,) * a.ndim)
    return pl.pallas_call(
        _stage_c_body,
        grid=grid,
        in_specs=[pl.BlockSpec((BLK, 16), lambda i: (i, 0)),
                  pl.BlockSpec((BLK, 16), lambda i: (i, 0)),
                  pl.BlockSpec((BLK, 8), lambda i: (i, 0)),
                  full(wnewt), full(pjw), full(pjb), full(eb2), full(bb2)],
        out_specs=pl.BlockSpec((BLK, 1), lambda i: (i, 0)),
        out_shape=jax.ShapeDtypeStruct((N, 1), F32),
    )(a2a, a2b, aux, wnewt, pjw, pjb, eb2, bb2)


def kernel(x, edge_index, enc_W1, enc_b1, enc_W2, enc_b2,
           env_g1_W, env_g1_as, env_g1_ad, env_g1_b,
           env_g2_W, env_g2_as, env_g2_ad, env_g2_b,
           beh_g1_W, beh_g1_as, beh_g1_ad, beh_g1_b,
           beh_g2_W, beh_g2_as, beh_g2_ad, beh_g2_b,
           W_gcn, rnn_Wih, rnn_bih, rnn_bhh, proj_W, proj_b,
           gate_W1, gate_b1, gate_W2, gate_b2):
    w1t = enc_W1.T
    w2t = enc_W2.T
    b1 = enc_b1.reshape(1, 16)
    b2 = enc_b2.reshape(1, 8)
    W3e = env_g1_W.reshape(2, 16, 11)
    W3b = beh_g1_W.reshape(2, 16, 11)
    afse = jnp.einsum('hc,hci->ih', env_g1_as, W3e)
    afde = jnp.einsum('hc,hci->ih', env_g1_ad, W3e)
    afsb = jnp.einsum('hc,hci->ih', beh_g1_as, W3b)
    afdb = jnp.einsum('hc,hci->ih', beh_g1_ad, W3b)
    Menv = jnp.zeros((22, 32), F32)
    Menv = Menv.at[0:11, 0:16].set(W3e[0].T).at[11:22, 16:32].set(W3e[1].T)
    Mbeh = jnp.zeros((22, 32), F32)
    Mbeh = Mbeh.at[0:11, 0:16].set(W3b[0].T).at[11:22, 16:32].set(W3b[1].T)
    benv = env_g1_b.reshape(1, 32)
    bbeh = beh_g1_b.reshape(1, 32)
    e2w = env_g2_W.T
    b2w = beh_g2_W.T
    eas = env_g2_as.reshape(1, 1)
    ead = env_g2_ad.reshape(1, 1)
    bas = beh_g2_as.reshape(1, 1)
    bad = beh_g2_ad.reshape(1, 1)
    gw1t = gate_W1.T
    gw2t = gate_W2.T
    gb1 = gate_b1.reshape(1, 32)
    gb2 = gate_b2.reshape(1, 3)
    wg = W_gcn.reshape(1, 100)
    wiht = rnn_Wih.T
    bsum = (rnn_bih + rnn_bhh).reshape(1, 100)
    pjw = proj_W.T
    pjb = proj_b.reshape(1, 1)
    eb2 = env_g2_b.reshape(1, 1)
    bb2 = beh_g2_b.reshape(1, 1)

    src = edge_index[0].astype(I32)
    dst = edge_index[1].astype(I32)
    padn = EPR * 128 - E
    s0 = jnp.concatenate([src, jnp.zeros((padn,), I32)]).reshape(EPR, 128)
    d0 = jnp.concatenate([dst, jnp.full((padn,), N, I32)]).reshape(EPR, 128)
    sidx2 = jnp.stack([s0, s0 + N])
    z24 = jnp.zeros((N_PAD, 24), F32)
    z16 = jnp.zeros((N_PAD, 16), F32)

    tab2, dsttab = _stage_a(x, w1t, b1, w2t, b2, afse, afde, afsb, afdb)
    stab = tab2.reshape(2 * N, 16)
    dtabp = jnp.concatenate([dsttab, jnp.zeros((8, 16), F32)], axis=0)

    out1 = _pass1(sidx2, d0, stab, dtabp, z24)
    accA = out1[0, :N]
    accB = out1[1, :N]

    src2, dst2, aux = _stage_b(accA, accB, tab2, x, Menv, benv, Mbeh, bbeh,
                               e2w, eas, ead, b2w, bas, bad,
                               gw1t, gb1, gw2t, gb2)
    d2tabp = jnp.concatenate([dst2, jnp.zeros((8, 16), F32)], axis=0)

    out2 = _pass2(s0, d0, src2, d2tabp, z16)

    wnewt = _evolve(wg, wiht, bsum).reshape(10, 10).T
    return _stage_c(out2[0, :N], out2[1, :N], aux, wnewt, pjw, pjb, eb2, bb2)


# speedup vs baseline: 221.9456x; 221.9456x over previous
import functools
import jax
import jax.numpy as jnp
from jax import lax
from jax.experimental import pallas as pl
from jax.experimental.pallas import tpu as pltpu, tpu_sc as plsc

N = 50000
E = 1600000
EPR = 12544
N_PAD = 50048
ZR = N_PAD // 16
BLK = 1000
F32 = jnp.float32
I32 = jnp.int32


def _lrelu(a):
    return jnp.where(a >= 0, a, 0.2 * a)


def _stage_a_body(x_ref, w1t, b1, w2t, b2, afse, afde, afsb, afdb,
                  tab_ref, dst_ref):
    xb = x_ref[...]
    h1 = jnp.maximum(jnp.dot(xb, w1t[...], preferred_element_type=F32)
                     + b1[...], 0.0)
    ctx = jnp.dot(h1, w2t[...], preferred_element_type=F32) + b2[...]
    xenv = jnp.concatenate([xb[:, 0:1], xb[:, 1:2], xb[:, 5:6], ctx], axis=1)
    xbeh = jnp.concatenate([xb[:, 0:1], xb[:, 1:2], xb[:, 2:3], ctx], axis=1)
    ase = jnp.dot(xenv, afse[...], preferred_element_type=F32)
    ade = jnp.dot(xenv, afde[...], preferred_element_type=F32)
    asb = jnp.dot(xbeh, afsb[...], preferred_element_type=F32)
    adb = jnp.dot(xbeh, afdb[...], preferred_element_type=F32)
    me = _lrelu(ase + ade)
    mb = _lrelu(asb + adb)
    z3 = jnp.zeros((xb.shape[0], 3), F32)
    tab_ref[0] = jnp.concatenate([xenv, ase, z3], axis=1)
    tab_ref[1] = jnp.concatenate([xbeh, asb, z3], axis=1)
    z8 = jnp.zeros((xb.shape[0], 8), F32)
    dst_ref[...] = jnp.concatenate([ade, me, adb, mb, z8], axis=1)


def _stage_a(x, w1t, b1, w2t, b2, afse, afde, afsb, afdb):
    grid = (N // BLK,)
    full = lambda a: pl.BlockSpec(a.shape, lambda i: (0,) * a.ndim)
    return pl.pallas_call(
        _stage_a_body,
        grid=grid,
        in_specs=[pl.BlockSpec((BLK, 6), lambda i: (i, 0)),
                  full(w1t), full(b1), full(w2t), full(b2),
                  full(afse), full(afde), full(afsb), full(afdb)],
        out_specs=[pl.BlockSpec((2, BLK, 16), lambda i: (0, i, 0)),
                   pl.BlockSpec((BLK, 16), lambda i: (i, 0))],
        out_shape=[jax.ShapeDtypeStruct((2, N, 16), F32),
                   jax.ShapeDtypeStruct((N, 16), F32)],
    )(x, w1t, b1, w2t, b2, afse, afde, afsb, afdb)


def _pass1_body(sidx2, didx, stab, dtab, zro, out,
                s_idx, d_idx, s_rows, d_rows, payl, accum, sem):
    cid = lax.axis_index("c")
    sid = lax.axis_index("s")
    pltpu.sync_copy(zro.at[pl.ds(sid * ZR, ZR)], accum.at[pl.ds(sid * ZR, ZR)])
    lane = lax.iota(I32, 16)
    ccol = [jnp.full((16,), c, I32) for c in range(24)]
    co = cid * 4
    dcol = [ccol[c] + co for c in range(4)]

    def chunk(i, carry):
        row0 = sid * 784 + i * 4
        pltpu.sync_copy(sidx2.at[cid, pl.ds(row0, 4)], s_idx)
        pltpu.sync_copy(didx.at[pl.ds(row0, 4)], d_idx)
        hs = []
        for j in range(4):
            hs.append(pltpu.async_copy(
                stab.at[s_idx.at[j]], s_rows.at[pl.ds(j * 128, 128)], sem))
            hs.append(pltpu.async_copy(
                dtab.at[d_idx.at[j]], d_rows.at[pl.ds(j * 128, 128)], sem))
        for h in hs:
            h.wait()

        def group(g, c2):
            r = g * 16 + lane
            feat = [plsc.load_gather(s_rows, [r, ccol[c]]) for c in range(11)]
            as0 = plsc.load_gather(s_rows, [r, ccol[11]])
            as1 = plsc.load_gather(s_rows, [r, ccol[12]])
            ad0 = plsc.load_gather(d_rows, [r, dcol[0]])
            ad1 = plsc.load_gather(d_rows, [r, dcol[1]])
            m0 = plsc.load_gather(d_rows, [r, dcol[2]])
            m1 = plsc.load_gather(d_rows, [r, dcol[3]])
            t0 = jnp.exp(_lrelu(as0 + ad0) - m0)
            t1 = jnp.exp(_lrelu(as1 + ad1) - m1)
            plsc.store_scatter(payl, [r, ccol[0]], t0)
            plsc.store_scatter(payl, [r, ccol[1]], t1)
            for c in range(11):
                plsc.store_scatter(payl, [r, ccol[2 + c]], t0 * feat[c])
                plsc.store_scatter(payl, [r, ccol[13 + c]], t1 * feat[c])
            return c2
        lax.fori_loop(0, 32, group, 0)
        for j in range(4):
            pltpu.sync_copy(payl.at[pl.ds(j * 128, 128)],
                            accum.at[d_idx.at[j]], add=True)
        return carry
    plsc.subcore_barrier()
    lax.fori_loop(0, 196, chunk, 0)
    plsc.subcore_barrier()
    pltpu.sync_copy(accum.at[pl.ds(sid * ZR, ZR)],
                    out.at[cid, pl.ds(sid * ZR, ZR)])


def _pass1(sidx2, didx, stab, dtab, zro):
    mesh = plsc.VectorSubcoreMesh(core_axis_name="c", subcore_axis_name="s")
    kern = functools.partial(
        pl.kernel,
        mesh=mesh,
        compiler_params=pltpu.CompilerParams(needs_layout_passes=False, use_tc_tiling_on_sc=False),
        out_type=jax.ShapeDtypeStruct((2, N_PAD, 24), F32),
        scratch_types=[
            pltpu.VMEM((4, 128), I32),
            pltpu.VMEM((4, 128), I32),
            pltpu.VMEM((512, 16), F32),
            pltpu.VMEM((512, 16), F32),
            pltpu.VMEM((512, 24), F32),
            pltpu.VMEM_SHARED((N_PAD, 24), F32),
            pltpu.SemaphoreType.DMA,
        ])(_pass1_body)
    return kern(sidx2, didx, stab, dtab, zro)


def _stage_b_body(accA, accB, tab, x_ref, Menv, benv, Mbeh, bbeh,
                  e2w, eas, ead, b2w, bas, bad, gw1t, gb1, gw2t, gb2,
                  src2_ref, dst2_ref, aux_ref):
    B = x_ref.shape[0]

    def layer1(acc, xf, M, b):
        den = acc[:, 0:2] + 1.0
        U = acc[:, 2:24] + jnp.concatenate([xf, xf], axis=1)
        d11 = jnp.concatenate(
            [jnp.broadcast_to(den[:, 0:1], (B, 11)),
             jnp.broadcast_to(den[:, 1:2], (B, 11))], axis=1) + 1e-16
        h = jnp.dot(U / d11, M[...], preferred_element_type=F32) + b[...]
        return jnp.where(h > 0, h, jnp.exp(jnp.minimum(h, 0.0)) - 1.0)

    xenv = tab[0][:, 0:11]
    xbeh = tab[1][:, 0:11]
    env_h = layer1(accA[...], xenv, Menv, benv)
    beh_h = layer1(accB[...], xbeh, Mbeh, bbeh)
    xl2e = jnp.dot(env_h, e2w[...], preferred_element_type=F32)
    xl2b = jnp.dot(beh_h, b2w[...], preferred_element_type=F32)
    as2e = xl2e * eas[0, 0]
    ad2e = xl2e * ead[0, 0]
    m2e = _lrelu(as2e + ad2e)
    as2b = xl2b * bas[0, 0]
    ad2b = xl2b * bad[0, 0]
    m2b = _lrelu(as2b + ad2b)
    ctx = tab[0][:, 3:11]
    g1 = jnp.maximum(jnp.dot(ctx, gw1t[...], preferred_element_type=F32)
                     + gb1[...], 0.0)
    g2 = jnp.dot(g1, gw2t[...], preferred_element_type=F32) + gb2[...]
    g2 = g2 - jnp.max(g2, axis=1, keepdims=True)
    eg = jnp.exp(g2)
    gate = eg / jnp.sum(eg, axis=1, keepdims=True)
    xb = x_ref[...]
    z2 = jnp.zeros((B, 2), F32)
    src2_ref[...] = jnp.concatenate(
        [xl2e, as2e, xl2b, as2b, xb[:, 3:4], xb[:, 4:5], ctx, z2], axis=1)
    dst2_ref[...] = jnp.concatenate(
        [ad2e, m2e, ad2b, m2b, jnp.zeros((B, 12), F32)], axis=1)
    aux_ref[...] = jnp.concatenate(
        [gate, xl2e, xl2b, jnp.zeros((B, 3), F32)], axis=1)


def _stage_b(accA, accB, tab2, x, Menv, benv, Mbeh, bbeh,
             e2w, eas, ead, b2w, bas, bad, gw1t, gb1, gw2t, gb2):
    grid = (N // BLK,)
    full = lambda a: pl.BlockSpec(a.shape, lambda i: (0,) * a.ndim)
    args = (accA, accB, tab2, x, Menv, benv, Mbeh, bbeh,
            e2w, eas, ead, b2w, bas, bad, gw1t, gb1, gw2t, gb2)
    return pl.pallas_call(
        _stage_b_body,
        grid=grid,
        in_specs=[pl.BlockSpec((BLK, 24), lambda i: (i, 0)),
                  pl.BlockSpec((BLK, 24), lambda i: (i, 0)),
                  pl.BlockSpec((2, BLK, 16), lambda i: (0, i, 0)),
                  pl.BlockSpec((BLK, 6), lambda i: (i, 0))] +
                 [full(a) for a in args[4:]],
        out_specs=[pl.BlockSpec((BLK, 16), lambda i: (i, 0)),
                   pl.BlockSpec((BLK, 16), lambda i: (i, 0)),
                   pl.BlockSpec((BLK, 8), lambda i: (i, 0))],
        out_shape=[jax.ShapeDtypeStruct((N, 16), F32),
                   jax.ShapeDtypeStruct((N, 16), F32),
                   jax.ShapeDtypeStruct((N, 8), F32)],
    )(*args)


def _pass2_body(sidx, didx, stab, dtab, zro, out,
                s_idx, d_idx, s_rows, d_rows, payl, accum, sem):
    cid = lax.axis_index("c")
    sid = lax.axis_index("s")
    pltpu.sync_copy(zro.at[pl.ds(sid * ZR, ZR)], accum.at[pl.ds(sid * ZR, ZR)])
    lane = lax.iota(I32, 16)
    ccol = [jnp.full((16,), c, I32) for c in range(16)]
    wid = sid * 2 + cid
    zv = jnp.zeros((16,), F32)

    def chunk(i, carry):
        row0 = wid * 392 + i * 8
        pltpu.sync_copy(sidx.at[pl.ds(row0, 8)], s_idx)
        pltpu.sync_copy(didx.at[pl.ds(row0, 8)], d_idx)
        hs = []
        for j in range(8):
            hs.append(pltpu.async_copy(
                stab.at[s_idx.at[j]], s_rows.at[pl.ds(j * 128, 128)], sem))
            hs.append(pltpu.async_copy(
                dtab.at[d_idx.at[j]], d_rows.at[pl.ds(j * 128, 128)], sem))
        for h in hs:
            h.wait()

        def group(g, c2):
            r = g * 16 + lane
            xl2e = plsc.load_gather(s_rows, [r, ccol[0]])
            as2e = plsc.load_gather(s_rows, [r, ccol[1]])
            xl2b = plsc.load_gather(s_rows, [r, ccol[2]])
            as2b = plsc.load_gather(s_rows, [r, ccol[3]])
            ad2e = plsc.load_gather(d_rows, [r, ccol[0]])
            m2e = plsc.load_gather(d_rows, [r, ccol[1]])
            ad2b = plsc.load_gather(d_rows, [r, ccol[2]])
            m2b = plsc.load_gather(d_rows, [r, ccol[3]])
            te = jnp.exp(_lrelu(as2e + ad2e) - m2e)
            tb = jnp.exp(_lrelu(as2b + ad2b) - m2b)
            plsc.store_scatter(payl, [r, ccol[0]], te)
            plsc.store_scatter(payl, [r, ccol[1]], te * xl2e)
            plsc.store_scatter(payl, [r, ccol[2]], tb)
            plsc.store_scatter(payl, [r, ccol[3]], tb * xl2b)
            for c in range(4, 14):
                v = plsc.load_gather(s_rows, [r, ccol[c]])
                plsc.store_scatter(payl, [r, ccol[c]], v)
            plsc.store_scatter(payl, [r, ccol[14]], zv)
            plsc.store_scatter(payl, [r, ccol[15]], zv)
            return c2
        lax.fori_loop(0, 64, group, 0)
        for j in range(8):
            pltpu.sync_copy(payl.at[pl.ds(j * 128, 128)],
                            accum.at[d_idx.at[j]], add=True)
        return carry
    plsc.subcore_barrier()
    lax.fori_loop(0, 49, chunk, 0)
    plsc.subcore_barrier()
    pltpu.sync_copy(accum.at[pl.ds(sid * ZR, ZR)],
                    out.at[cid, pl.ds(sid * ZR, ZR)])


def _pass2(sidx, didx, stab, dtab, zro):
    mesh = plsc.VectorSubcoreMesh(core_axis_name="c", subcore_axis_name="s")
    kern = functools.partial(
        pl.kernel,
        mesh=mesh,
        compiler_params=pltpu.CompilerParams(needs_layout_passes=False, use_tc_tiling_on_sc=False),
        out_type=jax.ShapeDtypeStruct((2, N_PAD, 16), F32),
        scratch_types=[
            pltpu.VMEM((8, 128), I32),
            pltpu.VMEM((8, 128), I32),
            pltpu.VMEM((1024, 16), F32),
            pltpu.VMEM((1024, 16), F32),
            pltpu.VMEM((1024, 16), F32),
            pltpu.VMEM_SHARED((N_PAD, 16), F32),
            pltpu.SemaphoreType.DMA,
        ])(_pass2_body)
    return kern(sidx, didx, stab, dtab, zro)


def _evolve_body(wg, wiht, bsum, out_ref):
    out_ref[...] = jnp.tanh(
        jnp.dot(wg[...], wiht[...], preferred_element_type=F32) + bsum[...])


def _evolve(wg, wiht, bsum):
    return pl.pallas_call(
        _evolve_body,
        out_shape=jax.ShapeDtypeStruct((1, 100), F32),
    )(wg, wiht, bsum)


def _stage_c_body(a2a, a2b, aux, wnewt, pjw, pjb, eb2, bb2, out_ref):
    a2 = a2a[...] + a2b[...]
    xl2e = aux[:, 3:4]
    xl2b = aux[:, 4:5]
    env_out = (a2[:, 1:2] + xl2e) / (a2[:, 0:1] + 1.0 + 1e-16) + eb2[0, 0]
    beh_out = (a2[:, 3:4] + xl2b) / (a2[:, 2:3] + 1.0 + 1e-16) + bb2[0, 0]
    aggf = a2[:, 4:14]
    ph = jnp.maximum(jnp.dot(aggf, wnewt[...], preferred_element_type=F32),
                     0.0)
    phys_out = jnp.dot(ph, pjw[...], preferred_element_type=F32) + pjb[0, 0]
    out_ref[...] = (aux[:, 0:1] * env_out + aux[:, 1:2] * phys_out
                    + aux[:, 2:3] * beh_out)


def _stage_c(a2a, a2b, aux, wnewt, pjw, pjb, eb2, bb2):
    grid = (N // BLK,)
    full = lambda a: pl.BlockSpec(a.shape, lambda i: (0,) * a.ndim)
    return pl.pallas_call(
        _stage_c_body,
        grid=grid,
        in_specs=[pl.BlockSpec((BLK, 16), lambda i: (i, 0)),
                  pl.BlockSpec((BLK, 16), lambda i: (i, 0)),
                  pl.BlockSpec((BLK, 8), lambda i: (i, 0)),
                  full(wnewt), full(pjw), full(pjb), full(eb2), full(bb2)],
        out_specs=pl.BlockSpec((BLK, 1), lambda i: (i, 0)),
        out_shape=jax.ShapeDtypeStruct((N, 1), F32),
    )(a2a, a2b, aux, wnewt, pjw, pjb, eb2, bb2)


def kernel(x, edge_index, enc_W1, enc_b1, enc_W2, enc_b2,
           env_g1_W, env_g1_as, env_g1_ad, env_g1_b,
           env_g2_W, env_g2_as, env_g2_ad, env_g2_b,
           beh_g1_W, beh_g1_as, beh_g1_ad, beh_g1_b,
           beh_g2_W, beh_g2_as, beh_g2_ad, beh_g2_b,
           W_gcn, rnn_Wih, rnn_bih, rnn_bhh, proj_W, proj_b,
           gate_W1, gate_b1, gate_W2, gate_b2):
    w1t = enc_W1.T
    w2t = enc_W2.T
    b1 = enc_b1.reshape(1, 16)
    b2 = enc_b2.reshape(1, 8)
    W3e = env_g1_W.reshape(2, 16, 11)
    W3b = beh_g1_W.reshape(2, 16, 11)
    afse = jnp.einsum('hc,hci->ih', env_g1_as, W3e)
    afde = jnp.einsum('hc,hci->ih', env_g1_ad, W3e)
    afsb = jnp.einsum('hc,hci->ih', beh_g1_as, W3b)
    afdb = jnp.einsum('hc,hci->ih', beh_g1_ad, W3b)
    Menv = jnp.zeros((22, 32), F32)
    Menv = Menv.at[0:11, 0:16].set(W3e[0].T).at[11:22, 16:32].set(W3e[1].T)
    Mbeh = jnp.zeros((22, 32), F32)
    Mbeh = Mbeh.at[0:11, 0:16].set(W3b[0].T).at[11:22, 16:32].set(W3b[1].T)
    benv = env_g1_b.reshape(1, 32)
    bbeh = beh_g1_b.reshape(1, 32)
    e2w = env_g2_W.T
    b2w = beh_g2_W.T
    eas = env_g2_as.reshape(1, 1)
    ead = env_g2_ad.reshape(1, 1)
    bas = beh_g2_as.reshape(1, 1)
    bad = beh_g2_ad.reshape(1, 1)
    gw1t = gate_W1.T
    gw2t = gate_W2.T
    gb1 = gate_b1.reshape(1, 32)
    gb2 = gate_b2.reshape(1, 3)
    wg = W_gcn.reshape(1, 100)
    wiht = rnn_Wih.T
    bsum = (rnn_bih + rnn_bhh).reshape(1, 100)
    pjw = proj_W.T
    pjb = proj_b.reshape(1, 1)
    eb2 = env_g2_b.reshape(1, 1)
    bb2 = beh_g2_b.reshape(1, 1)

    src = edge_index[0].astype(I32)
    dst = edge_index[1].astype(I32)
    padn = EPR * 128 - E
    s0 = jnp.concatenate([src, jnp.zeros((padn,), I32)]).reshape(EPR, 128)
    d0 = jnp.concatenate([dst, jnp.full((padn,), N, I32)]).reshape(EPR, 128)
    sidx2 = jnp.stack([s0, s0 + N])
    z24 = jnp.zeros((N_PAD, 24), F32)
    z16 = jnp.zeros((N_PAD, 16), F32)

    tab2, dsttab = _stage_a(x, w1t, b1, w2t, b2, afse, afde, afsb, afdb)
    stab = tab2.reshape(2 * N, 16)
    dtabp = jnp.concatenate([dsttab, jnp.zeros((8, 16), F32)], axis=0)

    out1 = _pass1(sidx2, d0, stab, dtabp, z24)
    accA = out1[0, :N]
    accB = out1[1, :N]

    src2, dst2, aux = _stage_b(accA, accB, tab2, x, Menv, benv, Mbeh, bbeh,
                               e2w, eas, ead, b2w, bas, bad,
                               gw1t, gb1, gw2t, gb2)
    d2tabp = jnp.concatenate([dst2, jnp.zeros((8, 16), F32)], axis=0)

    out2 = _pass2(s0, d0, src2, d2tabp, z16)

    wnewt = _evolve(wg, wiht, bsum).reshape(10, 10).T
    return _stage_c(out2[0, :N], out2[1, :N], aux, wnewt, pjw, pjb, eb2, bb2)
